# baseline (speedup 1.0000x reference)
import functools

import jax
import jax.numpy as jnp
from jax import lax
from jax.experimental import pallas as pl
from jax.experimental.pallas import tpu as pltpu
from jax.experimental.pallas import tpu_sc as plsc

_NC = 2
_NS = 16
_NW = _NC * _NS
_L = 16


def _masked_copy(s_slabs, n, chunk):
    mesh = plsc.VectorSubcoreMesh(core_axis_name="c", subcore_axis_name="s")

    @functools.partial(
        pl.kernel,
        out_type=jax.ShapeDtypeStruct((s_slabs * n,), jnp.float32),
        mesh=mesh,
        scratch_types=[
            pltpu.VMEM((_L,), jnp.int32),
            pltpu.VMEM((chunk,), jnp.float32),
            pltpu.SemaphoreType.DMA,
        ],
    )
    def body(x_hbm, m_hbm, out_hbm, m_v, zeros_v, sem):
        wid = lax.axis_index("s") * _NC + lax.axis_index("c")
        base = wid * chunk

        pltpu.sync_copy(m_hbm, m_v)
        mvec = m_v[...]

        def fill(i, _):
            zeros_v[pl.ds(i * _L, _L)] = jnp.zeros((_L,), jnp.float32)
            return 0

        lax.fori_loop(0, chunk // _L, fill, 0)

        for s in range(s_slabs):
            m = mvec[s]
            dst = out_hbm.at[pl.ds(s * n + base, chunk)]

            @pl.when(m != 0)
            def _copy():
                pltpu.async_copy(x_hbm.at[pl.ds(s * n + base, chunk)], dst, sem)

            @pl.when(m == 0)
            def _zero():
                pltpu.async_copy(zeros_v, dst, sem)

        for s in range(s_slabs):
            pltpu.make_async_copy(
                x_hbm.at[pl.ds(s * n + base, chunk)],
                out_hbm.at[pl.ds(s * n + base, chunk)],
                sem,
            ).wait()

    return body


def kernel(x, mask):
    B, K, H, W, Z = x.shape
    s_slabs = B * K
    n = H * W * Z
    chunk = n // _NW
    xf = x.reshape(s_slabs * n)
    m16 = jnp.zeros((_L,), jnp.int32).at[:s_slabs].set(
        mask.reshape(s_slabs).astype(jnp.int32))
    out = _masked_copy(s_slabs, n, chunk)(xf, m16)
    return out.reshape(B, K, H, W, Z)


# speedup vs baseline: 13.1435x; 13.1435x over previous
import functools

import jax
import jax.numpy as jnp
from jax import lax
from jax.experimental import pallas as pl
from jax.experimental.pallas import tpu as pltpu
from jax.experimental.pallas import tpu_sc as plsc

_NC = 2
_NS = 16
_NW = _NC * _NS
_L = 16
_NBUF = 6
_SPLIT = 4


def _masked_copy(s_slabs, n, chunk):
    half = chunk // _SPLIT
    nsub = _SPLIT * s_slabs
    mesh = plsc.VectorSubcoreMesh(core_axis_name="c", subcore_axis_name="s")

    @functools.partial(
        pl.kernel,
        out_type=jax.ShapeDtypeStruct((s_slabs * n,), jnp.float32),
        mesh=mesh,
        scratch_types=[
            pltpu.VMEM((_L,), jnp.int32),
            pltpu.VMEM((half,), jnp.float32),
            [pltpu.VMEM((half,), jnp.float32)] * _NBUF,
            [pltpu.SemaphoreType.DMA] * _NBUF,
            [pltpu.SemaphoreType.DMA] * _NBUF,
        ],
    )
    def body(x_hbm, m_hbm, out_hbm, m_v, zeros_v, bufs, gsem, ssem):
        wid = lax.axis_index("s") * _NC + lax.axis_index("c")
        base = wid * chunk

        pltpu.sync_copy(m_hbm, m_v)
        mvec = m_v[...]

        def src_at(i):
            s, h = i // _SPLIT, i % _SPLIT
            return x_hbm.at[pl.ds(s * n + base + h * half, half)]

        def dst_at(i):
            s, h = i // _SPLIT, i % _SPLIT
            return out_hbm.at[pl.ds(s * n + base + h * half, half)]

        for g in range(min(_NBUF - 1, nsub)):
            @pl.when(mvec[g // _SPLIT] != 0)
            def _pg():
                pltpu.async_copy(src_at(g), bufs[g % _NBUF], gsem[g % _NBUF])

        zvec = jnp.zeros((_L,), jnp.float32)

        def fill(i, _):
            for j in range(16):
                zeros_v[pl.ds((i * 16 + j) * _L, _L)] = zvec
            return 0

        lax.fori_loop(0, half // (_L * 16), fill, 0)

        for idx in range(nsub):
            g = idx + _NBUF - 1
            if g < nsub:
                bg = g % _NBUF
                if g >= _NBUF:
                    pltpu.make_async_copy(zeros_v, dst_at(g), ssem[bg]).wait()

                @pl.when(mvec[g // _SPLIT] != 0)
                def _gather():
                    pltpu.async_copy(src_at(g), bufs[bg], gsem[bg])

            b = idx % _NBUF
            m = mvec[idx // _SPLIT]

            @pl.when(m != 0)
            def _copy():
                pltpu.make_async_copy(src_at(idx), bufs[b], gsem[b]).wait()
                pltpu.async_copy(bufs[b], dst_at(idx), ssem[b])

            @pl.when(m == 0)
            def _zero():
                pltpu.async_copy(zeros_v, dst_at(idx), ssem[b])

        for idx in range(nsub - _NBUF, nsub):
            pltpu.make_async_copy(
                zeros_v, dst_at(idx), ssem[idx % _NBUF]).wait()

    return body


def kernel(x, mask):
    B, K, H, W, Z = x.shape
    s_slabs = B * K
    n = H * W * Z
    chunk = n // _NW
    xf = x.reshape(s_slabs * n)
    m16 = jnp.zeros((_L,), jnp.int32).at[:s_slabs].set(
        mask.reshape(s_slabs).astype(jnp.int32))
    out = _masked_copy(s_slabs, n, chunk)(xf, m16)
    return out.reshape(B, K, H, W, Z)
